# baseline (speedup 1.0000x reference)
import functools

import jax
import jax.numpy as jnp
from jax import lax
from jax.experimental import pallas as pl
from jax.experimental.pallas import tpu as pltpu
from jax.experimental.pallas import tpu_sc as plsc

_BATCH = 16384
_D = 64
_L = 16

_info = plsc.get_sparse_core_info()
_NC, _NS = _info.num_cores, _info.num_subcores
_NW = _NC * _NS
_BPW = _BATCH // _NW
_G = _BPW // _L
_CH = _D // _L
_FRPW = _BPW * _D // 128


@functools.partial(
    pl.kernel,
    mesh=plsc.VectorSubcoreMesh(core_axis_name="c", subcore_axis_name="s"),
    out_type=jax.ShapeDtypeStruct((_NW, _L), jnp.float32),
    scratch_types=[
        pltpu.VMEM((_BPW,), jnp.int32),
        pltpu.VMEM((_BPW,), jnp.int32),
        pltpu.VMEM((_FRPW, 128), jnp.float32),
        pltpu.VMEM((_BPW, 128), jnp.float32),
        pltpu.VMEM((_L,), jnp.float32),
        pltpu.SemaphoreType.DMA,
        pltpu.SemaphoreType.DMA,
    ],
    compiler_params=pltpu.CompilerParams(
        use_tc_tiling_on_sc=True, needs_layout_passes=False),
)
def _center_loss_sc(features_hbm, labels_hbm, centers_hbm, out_hbm,
                    lab_v, pidx_v, feat_v, rows_v, acc_v, gsem, fsem):
    wid = lax.axis_index("s") * _NC + lax.axis_index("c")
    base = wid * _BPW

    fcopy = pltpu.async_copy(
        features_hbm.at[pl.ds(wid * _FRPW, _FRPW)], feat_v, fsem)
    pltpu.sync_copy(labels_hbm.at[pl.ds(base, _BPW)], lab_v)

    def pbody(g, _):
        v = lab_v[pl.ds(g * _L, _L)]
        pidx_v[pl.ds(g * _L, _L)] = jnp.right_shift(v, 1)
        return 0

    lax.fori_loop(0, _G, pbody, 0)
    gcopy = pltpu.async_copy(centers_hbm.at[pidx_v], rows_v, gsem)
    fcopy.wait()
    gcopy.wait()

    zero = jnp.zeros((_L,), jnp.float32)

    def body(g, accs):
        lab16 = lab_v[pl.ds(g * _L, _L)]
        half = (lab16 & 1) * _D
        out = list(accs)
        for j in range(_L):
            i = g * _L + j
            h = half[j]
            for c in range(_CH):
                d = (feat_v[(i // 2), pl.ds((i % 2) * _D + c * _L, _L)]
                     - rows_v[i, pl.ds(h + c * _L, _L)])
                out[c] = out[c] + d * d
        return tuple(out)

    accs = lax.fori_loop(0, _G, body, (zero,) * _CH)
    acc_v[...] = (accs[0] + accs[1]) + (accs[2] + accs[3])
    pltpu.sync_copy(acc_v, out_hbm.at[wid])


def kernel(features, labels, centers):
    partials = _center_loss_sc(
        features.reshape(_BATCH * _D // 128, 128),
        labels.astype(jnp.int32),
        centers.reshape(-1, 128),
    )
    return jnp.sum(partials) * (0.5 / _BATCH)


# speedup vs baseline: 1.5655x; 1.5655x over previous
import functools

import jax
import jax.numpy as jnp
from jax import lax
from jax.experimental import pallas as pl
from jax.experimental.pallas import tpu as pltpu
from jax.experimental.pallas import tpu_sc as plsc

_BATCH = 16384
_D = 64
_NCLS = 100000
_L = 16

_info = plsc.get_sparse_core_info()
_NC, _NS = _info.num_cores, _info.num_subcores
_NW = _NC * _NS
_ROUNDS = _D // _NW
_LCH = 8192
_NLCH = _BATCH // _LCH


@functools.partial(
    pl.kernel,
    mesh=plsc.VectorSubcoreMesh(core_axis_name="c", subcore_axis_name="s"),
    out_type=jax.ShapeDtypeStruct((_NW, _L), jnp.float32),
    scratch_types=[
        pltpu.VMEM((_NCLS,), jnp.float32),
        pltpu.VMEM((_BATCH,), jnp.float32),
        pltpu.VMEM((_LCH,), jnp.int32),
        pltpu.VMEM((_L,), jnp.float32),
        pltpu.SemaphoreType.DMA,
        pltpu.SemaphoreType.DMA,
    ],
    compiler_params=pltpu.CompilerParams(
        use_tc_tiling_on_sc=True, needs_layout_passes=False),
)
def _center_loss_sc(features_hbm, labels_hbm, centers_hbm, out_hbm,
                    row_v, feat_v, lab_v, acc_v, rsem, fsem):
    wid = lax.axis_index("s") * _NC + lax.axis_index("c")

    zero = jnp.zeros((_L,), jnp.float32)
    accs = (zero, zero)

    for r in range(_ROUNDS):
        d = wid * _ROUNDS + r
        rcopy = pltpu.async_copy(centers_hbm.at[d, :], row_v, rsem)
        pltpu.sync_copy(features_hbm.at[d, :], feat_v)
        rcopy.wait()
        a0, a1 = accs
        for k in range(_NLCH):
            pltpu.sync_copy(labels_hbm.at[pl.ds(k * _LCH, _LCH)], lab_v)

            def body(v, acc2, _k=k):
                l16 = lab_v[pl.ds(v * _L, _L)]
                c = plsc.load_gather(row_v, [l16])
                f = feat_v[pl.ds(_k * _LCH + v * _L, _L)]
                d0 = f - c
                b0, b1 = acc2
                return (b1, b0 + d0 * d0)

            a0, a1 = lax.fori_loop(0, _LCH // _L, body, (a0, a1))
        accs = (a0, a1)

    acc_v[...] = accs[0] + accs[1]
    pltpu.sync_copy(acc_v, out_hbm.at[wid])


def kernel(features, labels, centers):
    partials = _center_loss_sc(
        features.T,
        labels.astype(jnp.int32),
        centers.T,
    )
    return jnp.sum(partials) * (0.5 / _BATCH)


# speedup vs baseline: 2.2665x; 1.4478x over previous
import functools

import jax
import jax.numpy as jnp
from jax import lax
from jax.experimental import pallas as pl
from jax.experimental.pallas import tpu as pltpu
from jax.experimental.pallas import tpu_sc as plsc

_BATCH = 16384
_D = 64
_NCLS = 100000
_L = 16

_info = plsc.get_sparse_core_info()
_NC, _NS = _info.num_cores, _info.num_subcores
_NW = _NC * _NS
_ROUNDS = _D // _NW
_LCH = 8192
_NLCH = _BATCH // _LCH


@functools.partial(
    pl.kernel,
    mesh=plsc.VectorSubcoreMesh(core_axis_name="c", subcore_axis_name="s"),
    out_type=jax.ShapeDtypeStruct((_NW, _L), jnp.float32),
    scratch_types=[
        pltpu.VMEM((_NCLS,), jnp.float32),
        pltpu.VMEM((_BATCH,), jnp.float32),
        pltpu.VMEM((_LCH,), jnp.int32),
        pltpu.VMEM((_L,), jnp.float32),
        pltpu.SemaphoreType.DMA,
        pltpu.SemaphoreType.DMA,
    ],
    compiler_params=pltpu.CompilerParams(
        use_tc_tiling_on_sc=True, needs_layout_passes=False),
)
def _center_loss_sc(features_hbm, labels_hbm, centers_hbm, out_hbm,
                    row_v, feat_v, lab_v, acc_v, rsem, fsem):
    wid = lax.axis_index("s") * _NC + lax.axis_index("c")

    zero = jnp.zeros((_L,), jnp.float32)
    accs = (zero, zero)

    for r in range(_ROUNDS):
        d = wid * _ROUNDS + r
        rcopy = pltpu.async_copy(centers_hbm.at[d, :], row_v, rsem)
        pltpu.sync_copy(features_hbm.at[d, :], feat_v)
        rcopy.wait()
        a0, a1 = accs
        for k in range(_NLCH):
            pltpu.sync_copy(labels_hbm.at[pl.ds(k * _LCH, _LCH)], lab_v)

            def body(v, acc2, _k=k):
                l16 = lab_v[pl.ds(v * _L, _L)]
                c = plsc.load_gather(row_v, [l16])
                f = feat_v[pl.ds(_k * _LCH + v * _L, _L)]
                d0 = f - c
                b0, b1 = acc2
                return (b1, b0 + d0 * d0)

            a0, a1 = lax.fori_loop(0, _LCH // _L, body, (a0, a1), unroll=8)
        accs = (a0, a1)

    acc_v[...] = accs[0] + accs[1]
    pltpu.sync_copy(acc_v, out_hbm.at[wid])


def kernel(features, labels, centers):
    partials = _center_loss_sc(
        features.T,
        labels.astype(jnp.int32),
        centers.T,
    )
    return jnp.sum(partials) * (0.5 / _BATCH)
